# baseline (speedup 1.0000x reference)
import functools

import jax
import jax.numpy as jnp
from jax import lax
from jax.experimental import pallas as pl
from jax.experimental.pallas import tpu as pltpu
from jax.experimental.pallas import tpu_sc as plsc

_NC = 2
_NS = 16
_NW = _NC * _NS
_CH = 128


@functools.lru_cache(maxsize=None)
def _make_sc_agg(sr, nch, d):
    assert nch % 2 == 0
    zc = sr // _NS // _CH
    mesh = plsc.VectorSubcoreMesh(core_axis_name="c", subcore_axis_name="s", num_cores=_NC, num_subcores=_NS)

    out_type = jax.ShapeDtypeStruct((_NC, sr, d), jnp.float32)
    scratch = [
        pltpu.VMEM((nch, _CH), jnp.int32),
        [pltpu.VMEM((1, _CH), jnp.int32)] * 2,
        [pltpu.VMEM((_CH, d), jnp.float32)] * 2,
        pltpu.VMEM_SHARED((sr, d), jnp.float32),
        [pltpu.SemaphoreType.DMA] * 2,
        [pltpu.SemaphoreType.DMA] * 2,
    ]

    def body(h_hbm, src_hbm, dst_hbm, z_hbm, agg_out,
             dst_v, srcb, rows, agg_sh, sem_s, sem_r):
        cid = lax.axis_index("c")
        sid = lax.axis_index("s")
        wid = cid * _NS + sid

        pltpu.sync_copy(dst_hbm.at[wid], dst_v)
        pltpu.sync_copy(z_hbm, rows[0])
        for k in range(zc):
            pltpu.sync_copy(rows[0], agg_sh.at[pl.ds((sid * zc + k) * _CH, _CH), :])
        plsc.subcore_barrier()

        pltpu.sync_copy(src_hbm.at[wid, pl.ds(0, 1)], srcb[0])
        pltpu.async_copy(h_hbm.at[srcb[0].at[0]], rows[0], sem_r[0])
        pltpu.async_copy(src_hbm.at[wid, pl.ds(1, 1)], srcb[1], sem_s[1])

        def wait_src(p):
            pltpu.make_async_copy(src_hbm.at[wid, pl.ds(0, 1)], srcb[p], sem_s[p]).wait()

        def wait_row(p):
            pltpu.make_async_copy(z_hbm, rows[p], sem_r[p]).wait()

        def pair(g, carry):
            for b in (0, 1):
                j = 2 * g + b
                p0, p1 = b, 1 - b
                if b == 0:
                    wait_src(p1)
                    pltpu.async_copy(h_hbm.at[srcb[p1].at[0]], rows[p1], sem_r[p1])
                else:
                    @pl.when(g < nch // 2 - 1)
                    def _():
                        wait_src(p1)
                        pltpu.async_copy(h_hbm.at[srcb[p1].at[0]], rows[p1], sem_r[p1])
                wait_row(p0)
                @pl.when(g < nch // 2 - 1)
                def _():
                    pltpu.async_copy(src_hbm.at[wid, pl.ds(j + 2, 1)], srcb[p0], sem_s[p0])
                pltpu.sync_copy(rows[p0], agg_sh.at[dst_v.at[j]], add=True)
            return carry

        lax.fori_loop(0, nch // 2, pair, 0)
        plsc.subcore_barrier()

        for k in range(zc):
            p = k % 2
            if k >= 2:
                pltpu.make_async_copy(z_hbm, rows[p], sem_r[p]).wait()
            r0 = (sid * zc + k) * _CH
            pltpu.sync_copy(agg_sh.at[pl.ds(r0, _CH), :], rows[p])
            pltpu.async_copy(rows[p], agg_out.at[cid, pl.ds(r0, _CH), :], sem_r[p])
        for k in range(max(zc - 2, 0), zc):
            pltpu.make_async_copy(z_hbm, rows[k % 2], sem_r[k % 2]).wait()

    return pl.kernel(body, out_type=out_type, mesh=mesh, scratch_types=scratch)


@functools.lru_cache(maxsize=None)
def _make_sc_deg(sr, nch, d):
    zc = sr // _NS // _CH
    mesh = plsc.VectorSubcoreMesh(core_axis_name="c", subcore_axis_name="s", num_cores=_NC, num_subcores=_NS)

    out_type = jax.ShapeDtypeStruct((_NC, sr, d), jnp.float32)
    scratch = [
        pltpu.VMEM((nch, _CH), jnp.int32),
        pltpu.VMEM((_CH, d), jnp.float32),
        pltpu.VMEM((_CH, d), jnp.float32),
        pltpu.VMEM_SHARED((sr, d), jnp.float32),
    ]

    def body(dst_hbm, z_hbm, ones_hbm, deg_out, dst_v, ones_v, zb_v, deg_sh):
        cid = lax.axis_index("c")
        sid = lax.axis_index("s")
        wid = cid * _NS + sid

        pltpu.sync_copy(dst_hbm.at[wid], dst_v)
        pltpu.sync_copy(z_hbm, zb_v)
        for k in range(zc):
            pltpu.sync_copy(zb_v, deg_sh.at[pl.ds((sid * zc + k) * _CH, _CH), :])
        pltpu.sync_copy(ones_hbm, ones_v)
        plsc.subcore_barrier()

        def step(j, carry):
            pltpu.sync_copy(ones_v, deg_sh.at[dst_v.at[j]], add=True)
            return carry

        lax.fori_loop(0, nch, step, 0)
        plsc.subcore_barrier()

        for k in range(zc):
            r0 = (sid * zc + k) * _CH
            pltpu.sync_copy(deg_sh.at[pl.ds(r0, _CH), :], zb_v)
            pltpu.sync_copy(zb_v, deg_out.at[cid, pl.ds(r0, _CH), :])

    return pl.kernel(body, out_type=out_type, mesh=mesh, scratch_types=scratch)


_TC_PARAMS = pltpu.CompilerParams(vmem_limit_bytes=100 * 1024 * 1024)


def _emb(x, w, b):
    def body(x_ref, w_ref, b_ref, o_ref):
        o_ref[...] = (
            jnp.dot(x_ref[...], w_ref[...], preferred_element_type=jnp.float32)
            + b_ref[...]
        )

    return pl.pallas_call(
        body,
        out_shape=jax.ShapeDtypeStruct((x.shape[0], w.shape[1]), jnp.float32),
        compiler_params=_TC_PARAMS,
    )(x, w, b)


def _layer_math(h, ap_ref, dp_ref, ws, wn, bias, gam, bet, n):
    psum = ap_ref[0, :n, :] + ap_ref[1, :n, :]
    deg = jnp.maximum(dp_ref[0, :n, 0:1] + dp_ref[1, :n, 0:1], 1.0)
    agg = psum / deg
    hh = (
        jnp.dot(h, ws, preferred_element_type=jnp.float32)
        + jnp.dot(agg, wn, preferred_element_type=jnp.float32)
        + bias
    )
    mean = jnp.mean(hh, axis=0, keepdims=True)
    c = hh - mean
    var = jnp.mean(c * c, axis=0, keepdims=True)
    hh = gam * c * lax.rsqrt(var + 1e-5) + bet
    return h + jnp.maximum(hh, 0.0)


def _layer(h, agg_p, deg_p, ws, wn, bias, gam, bet):
    n = h.shape[0]

    def body(h_ref, ap_ref, dp_ref, ws_ref, wn_ref, b_ref, g_ref, be_ref, o_ref):
        o_ref[...] = _layer_math(
            h_ref[...], ap_ref, dp_ref, ws_ref[...], wn_ref[...],
            b_ref[...], g_ref[...], be_ref[...], n,
        )

    return pl.pallas_call(
        body,
        out_shape=jax.ShapeDtypeStruct(h.shape, jnp.float32),
        compiler_params=_TC_PARAMS,
    )(h, agg_p, deg_p, ws, wn, bias, gam, bet)


def _final(h, agg_p, deg_p, ws, wn, bias, gam, bet, batch2d, ng, mlp):
    n = h.shape[0]
    nc_out = mlp[2][0].shape[1]

    def body(h_ref, ap_ref, dp_ref, ws_ref, wn_ref, b_ref, g_ref, be_ref,
             batch_ref, w1_ref, b1_ref, w2_ref, b2_ref, w3_ref, b3_ref, o_ref):
        h4 = _layer_math(
            h_ref[...], ap_ref, dp_ref, ws_ref[...], wn_ref[...],
            b_ref[...], g_ref[...], be_ref[...], n,
        )
        groups = lax.broadcasted_iota(jnp.int32, (ng, n), 0)
        m = (batch_ref[...] == groups).astype(jnp.float32)
        g = jnp.dot(m, h4, preferred_element_type=jnp.float32)
        g = jnp.maximum(
            jnp.dot(g, w1_ref[...], preferred_element_type=jnp.float32)
            + b1_ref[...], 0.0)
        g = jnp.maximum(
            jnp.dot(g, w2_ref[...], preferred_element_type=jnp.float32)
            + b2_ref[...], 0.0)
        o_ref[...] = (
            jnp.dot(g, w3_ref[...], preferred_element_type=jnp.float32)
            + b3_ref[...]
        )

    return pl.pallas_call(
        body,
        out_shape=jax.ShapeDtypeStruct((ng, nc_out), jnp.float32),
        compiler_params=_TC_PARAMS,
    )(h, agg_p, deg_p, ws, wn, bias, gam, bet, batch2d,
      mlp[0][0], mlp[0][1].reshape(1, -1),
      mlp[1][0], mlp[1][1].reshape(1, -1),
      mlp[2][0], mlp[2][1].reshape(1, -1))


def kernel(x, edge_index, batch, params):
    n, _ = x.shape
    e = edge_index.shape[1]
    d = params["emb_W"].shape[1]
    ng = 64

    ept = -(-e // (_NW * 2 * _CH)) * 2 * _CH
    nch = ept // _CH
    epad = _NW * ept - e
    src = jnp.concatenate([edge_index[0], jnp.zeros((epad,), jnp.int32)])
    dst = jnp.concatenate([edge_index[1], jnp.full((epad,), n, jnp.int32)])
    src3 = src.reshape(_NW, nch, _CH)
    dst3 = dst.reshape(_NW, nch, _CH)

    sr = -(-(n + 1) // (_NS * _CH)) * (_NS * _CH)
    z128 = jnp.zeros((_CH, d), jnp.float32)
    ones128 = jnp.ones((_CH, d), jnp.float32)

    h = _emb(x, params["emb_W"], params["emb_b"].reshape(1, -1))

    sc = _make_sc_agg(sr, nch, d)
    deg_p = _make_sc_deg(sr, nch, d)(dst3, z128, ones128)
    batch2d = batch.reshape(1, n)

    out = None
    for li, lp in enumerate(params["layers"]):
        bias = (lp["b_self"] + lp["b_neigh"]).reshape(1, -1)
        gam = lp["gamma"].reshape(1, -1)
        bet = lp["beta"].reshape(1, -1)
        agg_p = sc(h, src3, dst3, z128)
        if li < len(params["layers"]) - 1:
            h = _layer(h, agg_p, deg_p, lp["W_self"], lp["W_neigh"], bias, gam, bet)
        else:
            out = _final(h, agg_p, deg_p, lp["W_self"], lp["W_neigh"], bias,
                         gam, bet, batch2d, ng, params["mlp"])
    return out


# speedup vs baseline: 1.0002x; 1.0002x over previous
import functools

import jax
import jax.numpy as jnp
from jax import lax
from jax.experimental import pallas as pl
from jax.experimental.pallas import tpu as pltpu
from jax.experimental.pallas import tpu_sc as plsc

_NC = 2
_NS = 16
_NW = _NC * _NS
_CH = 128


@functools.lru_cache(maxsize=None)
def _make_sc_agg(sr, nch, d):
    assert nch % 2 == 0
    zc = sr // _NS // _CH
    mesh = plsc.VectorSubcoreMesh(core_axis_name="c", subcore_axis_name="s", num_cores=_NC, num_subcores=_NS)

    out_type = jax.ShapeDtypeStruct((_NC, sr, d), jnp.float32)
    scratch = [
        pltpu.VMEM((nch, _CH), jnp.int32),
        [pltpu.VMEM((1, _CH), jnp.int32)] * 2,
        [pltpu.VMEM((_CH, d), jnp.float32)] * 2,
        pltpu.VMEM_SHARED((sr, d), jnp.float32),
        [pltpu.SemaphoreType.DMA] * 2,
        [pltpu.SemaphoreType.DMA] * 2,
    ]

    def body(h_hbm, src_hbm, dst_hbm, z_hbm, agg_out,
             src_v, dstb, rows, agg_sh, sem_d, sem_r):
        cid = lax.axis_index("c")
        sid = lax.axis_index("s")
        wid = cid * _NS + sid

        pltpu.sync_copy(src_hbm.at[wid], src_v)
        pltpu.sync_copy(z_hbm, rows[0])
        for k in range(zc):
            pltpu.sync_copy(rows[0], agg_sh.at[pl.ds((sid * zc + k) * _CH, _CH), :])
        plsc.subcore_barrier()

        pltpu.async_copy(h_hbm.at[src_v.at[0]], rows[0], sem_r[0])
        pltpu.async_copy(dst_hbm.at[wid, pl.ds(0, 1)], dstb[0], sem_d[0])
        pltpu.async_copy(dst_hbm.at[wid, pl.ds(1, 1)], dstb[1], sem_d[1])

        def wait_dst(p):
            pltpu.make_async_copy(dst_hbm.at[wid, pl.ds(0, 1)], dstb[p], sem_d[p]).wait()

        def wait_row(p):
            pltpu.make_async_copy(z_hbm, rows[p], sem_r[p]).wait()

        def pair(g, carry):
            for b in (0, 1):
                j = 2 * g + b
                p0, p1 = b, 1 - b
                if b == 0:
                    pltpu.async_copy(h_hbm.at[src_v.at[j + 1]], rows[p1], sem_r[p1])
                else:
                    @pl.when(g < nch // 2 - 1)
                    def _():
                        pltpu.async_copy(h_hbm.at[src_v.at[j + 1]], rows[p1], sem_r[p1])
                wait_row(p0)
                wait_dst(p0)
                pltpu.sync_copy(rows[p0], agg_sh.at[dstb[p0].at[0]], add=True)
                @pl.when(g < nch // 2 - 1)
                def _():
                    pltpu.async_copy(dst_hbm.at[wid, pl.ds(j + 2, 1)], dstb[p0], sem_d[p0])
            return carry

        lax.fori_loop(0, nch // 2, pair, 0)
        plsc.subcore_barrier()

        for k in range(zc):
            p = k % 2
            if k >= 2:
                pltpu.make_async_copy(z_hbm, rows[p], sem_r[p]).wait()
            r0 = (sid * zc + k) * _CH
            pltpu.sync_copy(agg_sh.at[pl.ds(r0, _CH), :], rows[p])
            pltpu.async_copy(rows[p], agg_out.at[cid, pl.ds(r0, _CH), :], sem_r[p])
        for k in range(max(zc - 2, 0), zc):
            pltpu.make_async_copy(z_hbm, rows[k % 2], sem_r[k % 2]).wait()

    return pl.kernel(body, out_type=out_type, mesh=mesh, scratch_types=scratch)


@functools.lru_cache(maxsize=None)
def _make_sc_deg(sr, nch, d):
    zc = sr // _NS // _CH
    mesh = plsc.VectorSubcoreMesh(core_axis_name="c", subcore_axis_name="s", num_cores=_NC, num_subcores=_NS)

    out_type = jax.ShapeDtypeStruct((_NC, sr, d), jnp.float32)
    scratch = [
        pltpu.VMEM((nch, _CH), jnp.int32),
        pltpu.VMEM((_CH, d), jnp.float32),
        pltpu.VMEM((_CH, d), jnp.float32),
        pltpu.VMEM_SHARED((sr, d), jnp.float32),
    ]

    def body(dst_hbm, z_hbm, ones_hbm, deg_out, dst_v, ones_v, zb_v, deg_sh):
        cid = lax.axis_index("c")
        sid = lax.axis_index("s")
        wid = cid * _NS + sid

        pltpu.sync_copy(dst_hbm.at[wid], dst_v)
        pltpu.sync_copy(z_hbm, zb_v)
        for k in range(zc):
            pltpu.sync_copy(zb_v, deg_sh.at[pl.ds((sid * zc + k) * _CH, _CH), :])
        pltpu.sync_copy(ones_hbm, ones_v)
        plsc.subcore_barrier()

        def step(j, carry):
            pltpu.sync_copy(ones_v, deg_sh.at[dst_v.at[j]], add=True)
            return carry

        lax.fori_loop(0, nch, step, 0)
        plsc.subcore_barrier()

        for k in range(zc):
            r0 = (sid * zc + k) * _CH
            pltpu.sync_copy(deg_sh.at[pl.ds(r0, _CH), :], zb_v)
            pltpu.sync_copy(zb_v, deg_out.at[cid, pl.ds(r0, _CH), :])

    return pl.kernel(body, out_type=out_type, mesh=mesh, scratch_types=scratch)


_TC_PARAMS = pltpu.CompilerParams(vmem_limit_bytes=100 * 1024 * 1024)


def _emb(x, w, b):
    def body(x_ref, w_ref, b_ref, o_ref):
        o_ref[...] = (
            jnp.dot(x_ref[...], w_ref[...], preferred_element_type=jnp.float32)
            + b_ref[...]
        )

    return pl.pallas_call(
        body,
        out_shape=jax.ShapeDtypeStruct((x.shape[0], w.shape[1]), jnp.float32),
        compiler_params=_TC_PARAMS,
    )(x, w, b)


def _layer_math(h, ap_ref, dp_ref, ws, wn, bias, gam, bet, n):
    psum = ap_ref[0, :n, :] + ap_ref[1, :n, :]
    deg = jnp.maximum(dp_ref[0, :n, 0:1] + dp_ref[1, :n, 0:1], 1.0)
    agg = psum / deg
    hh = (
        jnp.dot(h, ws, preferred_element_type=jnp.float32)
        + jnp.dot(agg, wn, preferred_element_type=jnp.float32)
        + bias
    )
    mean = jnp.mean(hh, axis=0, keepdims=True)
    c = hh - mean
    var = jnp.mean(c * c, axis=0, keepdims=True)
    hh = gam * c * lax.rsqrt(var + 1e-5) + bet
    return h + jnp.maximum(hh, 0.0)


def _layer(h, agg_p, deg_p, ws, wn, bias, gam, bet):
    n = h.shape[0]

    def body(h_ref, ap_ref, dp_ref, ws_ref, wn_ref, b_ref, g_ref, be_ref, o_ref):
        o_ref[...] = _layer_math(
            h_ref[...], ap_ref, dp_ref, ws_ref[...], wn_ref[...],
            b_ref[...], g_ref[...], be_ref[...], n,
        )

    return pl.pallas_call(
        body,
        out_shape=jax.ShapeDtypeStruct(h.shape, jnp.float32),
        compiler_params=_TC_PARAMS,
    )(h, agg_p, deg_p, ws, wn, bias, gam, bet)


def _final(h, agg_p, deg_p, ws, wn, bias, gam, bet, batch2d, ng, mlp):
    n = h.shape[0]
    nc_out = mlp[2][0].shape[1]

    def body(h_ref, ap_ref, dp_ref, ws_ref, wn_ref, b_ref, g_ref, be_ref,
             batch_ref, w1_ref, b1_ref, w2_ref, b2_ref, w3_ref, b3_ref, o_ref):
        h4 = _layer_math(
            h_ref[...], ap_ref, dp_ref, ws_ref[...], wn_ref[...],
            b_ref[...], g_ref[...], be_ref[...], n,
        )
        groups = lax.broadcasted_iota(jnp.int32, (ng, n), 0)
        m = (batch_ref[...] == groups).astype(jnp.float32)
        g = jnp.dot(m, h4, preferred_element_type=jnp.float32)
        g = jnp.maximum(
            jnp.dot(g, w1_ref[...], preferred_element_type=jnp.float32)
            + b1_ref[...], 0.0)
        g = jnp.maximum(
            jnp.dot(g, w2_ref[...], preferred_element_type=jnp.float32)
            + b2_ref[...], 0.0)
        o_ref[...] = (
            jnp.dot(g, w3_ref[...], preferred_element_type=jnp.float32)
            + b3_ref[...]
        )

    return pl.pallas_call(
        body,
        out_shape=jax.ShapeDtypeStruct((ng, nc_out), jnp.float32),
        compiler_params=_TC_PARAMS,
    )(h, agg_p, deg_p, ws, wn, bias, gam, bet, batch2d,
      mlp[0][0], mlp[0][1].reshape(1, -1),
      mlp[1][0], mlp[1][1].reshape(1, -1),
      mlp[2][0], mlp[2][1].reshape(1, -1))


def kernel(x, edge_index, batch, params):
    n, _ = x.shape
    e = edge_index.shape[1]
    d = params["emb_W"].shape[1]
    ng = 64

    ept = -(-e // (_NW * 2 * _CH)) * 2 * _CH
    nch = ept // _CH
    epad = _NW * ept - e
    src = jnp.concatenate([edge_index[0], jnp.zeros((epad,), jnp.int32)])
    dst = jnp.concatenate([edge_index[1], jnp.full((epad,), n, jnp.int32)])
    src3 = src.reshape(_NW, nch, _CH)
    dst3 = dst.reshape(_NW, nch, _CH)

    sr = -(-(n + 1) // (_NS * _CH)) * (_NS * _CH)
    z128 = jnp.zeros((_CH, d), jnp.float32)
    ones128 = jnp.ones((_CH, d), jnp.float32)

    h = _emb(x, params["emb_W"], params["emb_b"].reshape(1, -1))

    sc = _make_sc_agg(sr, nch, d)
    deg_p = _make_sc_deg(sr, nch, d)(dst3, z128, ones128)
    batch2d = batch.reshape(1, n)

    out = None
    for li, lp in enumerate(params["layers"]):
        bias = (lp["b_self"] + lp["b_neigh"]).reshape(1, -1)
        gam = lp["gamma"].reshape(1, -1)
        bet = lp["beta"].reshape(1, -1)
        agg_p = sc(h, src3, dst3, z128)
        if li < len(params["layers"]) - 1:
            h = _layer(h, agg_p, deg_p, lp["W_self"], lp["W_neigh"], bias, gam, bet)
        else:
            out = _final(h, agg_p, deg_p, lp["W_self"], lp["W_neigh"], bias,
                         gam, bet, batch2d, ng, params["mlp"])
    return out


# speedup vs baseline: 1.5817x; 1.5813x over previous
import functools

import jax
import jax.numpy as jnp
from jax import lax
from jax.experimental import pallas as pl
from jax.experimental.pallas import tpu as pltpu
from jax.experimental.pallas import tpu_sc as plsc

_NC = 2
_NS = 16
_NW = _NC * _NS
_CH = 128
_SPLIT0 = 0.5


@functools.lru_cache(maxsize=None)
def _make_sc_agg(sr, nch0, nch1, d):
    assert nch0 % 2 == 0 and nch1 % 2 == 0
    nchm = max(nch0, nch1)
    zc = sr // _NS // _CH
    mesh = plsc.VectorSubcoreMesh(core_axis_name="c", subcore_axis_name="s", num_cores=_NC, num_subcores=_NS)

    out_type = jax.ShapeDtypeStruct((_NC, sr, d), jnp.float32)
    scratch = [
        pltpu.VMEM((nchm, _CH), jnp.int32),
        [pltpu.VMEM((1, _CH), jnp.int32)] * 2,
        [pltpu.VMEM((_CH, d), jnp.float32)] * 2,
        pltpu.VMEM_SHARED((sr, d), jnp.float32),
        [pltpu.SemaphoreType.DMA] * 2,
        [pltpu.SemaphoreType.DMA] * 2,
    ]

    def body(h_hbm, src_hbm, dst_hbm, z_hbm, agg_out,
             src_v, dstb, rows, agg_sh, sem_d, sem_r):
        cid = lax.axis_index("c")
        sid = lax.axis_index("s")
        wid = cid * _NS + sid
        gbound = jnp.where(cid == 0, nch0 // 2, nch1 // 2)

        pltpu.sync_copy(src_hbm.at[wid], src_v)
        pltpu.sync_copy(z_hbm, rows[0])
        for k in range(zc):
            pltpu.sync_copy(rows[0], agg_sh.at[pl.ds((sid * zc + k) * _CH, _CH), :])
        plsc.subcore_barrier()

        pltpu.async_copy(h_hbm.at[src_v.at[0]], rows[0], sem_r[0])
        pltpu.async_copy(dst_hbm.at[wid, pl.ds(0, 1)], dstb[0], sem_d[0])
        pltpu.async_copy(dst_hbm.at[wid, pl.ds(1, 1)], dstb[1], sem_d[1])

        def wait_dst(p):
            pltpu.make_async_copy(dst_hbm.at[wid, pl.ds(0, 1)], dstb[p], sem_d[p]).wait()

        def wait_row(p):
            pltpu.make_async_copy(z_hbm, rows[p], sem_r[p]).wait()

        def pair(g, carry):
            for b in (0, 1):
                j = 2 * g + b
                p0, p1 = b, 1 - b
                if b == 0:
                    pltpu.async_copy(h_hbm.at[src_v.at[j + 1]], rows[p1], sem_r[p1])
                else:
                    @pl.when(g < gbound - 1)
                    def _():
                        pltpu.async_copy(h_hbm.at[src_v.at[j + 1]], rows[p1], sem_r[p1])
                wait_row(p0)
                wait_dst(p0)
                pltpu.sync_copy(rows[p0], agg_sh.at[dstb[p0].at[0]], add=True)
                @pl.when(g < gbound - 1)
                def _():
                    pltpu.async_copy(dst_hbm.at[wid, pl.ds(j + 2, 1)], dstb[p0], sem_d[p0])
            return carry

        lax.fori_loop(0, gbound, pair, 0)
        plsc.subcore_barrier()

        for k in range(zc):
            p = k % 2
            if k >= 2:
                pltpu.make_async_copy(z_hbm, rows[p], sem_r[p]).wait()
            r0 = (sid * zc + k) * _CH
            pltpu.sync_copy(agg_sh.at[pl.ds(r0, _CH), :], rows[p])
            pltpu.async_copy(rows[p], agg_out.at[cid, pl.ds(r0, _CH), :], sem_r[p])
        for k in range(max(zc - 2, 0), zc):
            pltpu.make_async_copy(z_hbm, rows[k % 2], sem_r[k % 2]).wait()

    return pl.kernel(body, out_type=out_type, mesh=mesh, scratch_types=scratch)


@functools.lru_cache(maxsize=None)
def _make_sc_deg(sr, nch0, nch1, d):
    zc = sr // _NS // _CH
    nchm = max(nch0, nch1)
    mesh = plsc.VectorSubcoreMesh(core_axis_name="c", subcore_axis_name="s", num_cores=_NC, num_subcores=_NS)

    out_type = jax.ShapeDtypeStruct((_NC, sr, d), jnp.float32)
    scratch = [
        pltpu.VMEM((nchm, _CH), jnp.int32),
        pltpu.VMEM((_CH, d), jnp.float32),
        pltpu.VMEM((_CH, d), jnp.float32),
        pltpu.VMEM_SHARED((sr, d), jnp.float32),
    ]

    def body(dst_hbm, z_hbm, ones_hbm, deg_out, dst_v, ones_v, zb_v, deg_sh):
        cid = lax.axis_index("c")
        sid = lax.axis_index("s")
        wid = cid * _NS + sid
        bound = jnp.where(cid == 0, nch0, nch1)

        pltpu.sync_copy(dst_hbm.at[wid], dst_v)
        pltpu.sync_copy(z_hbm, zb_v)
        for k in range(zc):
            pltpu.sync_copy(zb_v, deg_sh.at[pl.ds((sid * zc + k) * _CH, _CH), :])
        pltpu.sync_copy(ones_hbm, ones_v)
        plsc.subcore_barrier()

        def step(j, carry):
            pltpu.sync_copy(ones_v, deg_sh.at[dst_v.at[j]], add=True)
            return carry

        lax.fori_loop(0, bound, step, 0)
        plsc.subcore_barrier()

        for k in range(zc):
            r0 = (sid * zc + k) * _CH
            pltpu.sync_copy(deg_sh.at[pl.ds(r0, _CH), :], zb_v)
            pltpu.sync_copy(zb_v, deg_out.at[cid, pl.ds(r0, _CH), :])

    return pl.kernel(body, out_type=out_type, mesh=mesh, scratch_types=scratch)


_TC_PARAMS = pltpu.CompilerParams(vmem_limit_bytes=100 * 1024 * 1024)


def _emb(x, w, b):
    def body(x_ref, w_ref, b_ref, o_ref):
        o_ref[...] = (
            jnp.dot(x_ref[...], w_ref[...], preferred_element_type=jnp.float32)
            + b_ref[...]
        )

    return pl.pallas_call(
        body,
        out_shape=jax.ShapeDtypeStruct((x.shape[0], w.shape[1]), jnp.float32),
        compiler_params=_TC_PARAMS,
    )(x, w, b)


def _layer_math(h, ap_ref, dp_ref, ws, wn, bias, gam, bet, n):
    psum = ap_ref[0, :n, :] + ap_ref[1, :n, :]
    deg = jnp.maximum(dp_ref[0, :n, 0:1] + dp_ref[1, :n, 0:1], 1.0)
    agg = psum / deg
    hh = (
        jnp.dot(h, ws, preferred_element_type=jnp.float32)
        + jnp.dot(agg, wn, preferred_element_type=jnp.float32)
        + bias
    )
    mean = jnp.mean(hh, axis=0, keepdims=True)
    c = hh - mean
    var = jnp.mean(c * c, axis=0, keepdims=True)
    hh = gam * c * lax.rsqrt(var + 1e-5) + bet
    return h + jnp.maximum(hh, 0.0)


def _layer(h, agg_p, deg_p, ws, wn, bias, gam, bet):
    n = h.shape[0]

    def body(h_ref, ap_ref, dp_ref, ws_ref, wn_ref, b_ref, g_ref, be_ref, o_ref):
        o_ref[...] = _layer_math(
            h_ref[...], ap_ref, dp_ref, ws_ref[...], wn_ref[...],
            b_ref[...], g_ref[...], be_ref[...], n,
        )

    return pl.pallas_call(
        body,
        out_shape=jax.ShapeDtypeStruct(h.shape, jnp.float32),
        compiler_params=_TC_PARAMS,
    )(h, agg_p, deg_p, ws, wn, bias, gam, bet)


def _final(h, agg_p, deg_p, ws, wn, bias, gam, bet, batch2d, ng, mlp):
    n = h.shape[0]
    nc_out = mlp[2][0].shape[1]

    def body(h_ref, ap_ref, dp_ref, ws_ref, wn_ref, b_ref, g_ref, be_ref,
             batch_ref, w1_ref, b1_ref, w2_ref, b2_ref, w3_ref, b3_ref, o_ref):
        h4 = _layer_math(
            h_ref[...], ap_ref, dp_ref, ws_ref[...], wn_ref[...],
            b_ref[...], g_ref[...], be_ref[...], n,
        )
        groups = lax.broadcasted_iota(jnp.int32, (ng, n), 0)
        m = (batch_ref[...] == groups).astype(jnp.float32)
        g = jnp.dot(m, h4, preferred_element_type=jnp.float32)
        g = jnp.maximum(
            jnp.dot(g, w1_ref[...], preferred_element_type=jnp.float32)
            + b1_ref[...], 0.0)
        g = jnp.maximum(
            jnp.dot(g, w2_ref[...], preferred_element_type=jnp.float32)
            + b2_ref[...], 0.0)
        o_ref[...] = (
            jnp.dot(g, w3_ref[...], preferred_element_type=jnp.float32)
            + b3_ref[...]
        )

    return pl.pallas_call(
        body,
        out_shape=jax.ShapeDtypeStruct((ng, nc_out), jnp.float32),
        compiler_params=_TC_PARAMS,
    )(h, agg_p, deg_p, ws, wn, bias, gam, bet, batch2d,
      mlp[0][0], mlp[0][1].reshape(1, -1),
      mlp[1][0], mlp[1][1].reshape(1, -1),
      mlp[2][0], mlp[2][1].reshape(1, -1))


def kernel(x, edge_index, batch, params):
    n, _ = x.shape
    e = edge_index.shape[1]
    d = params["emb_W"].shape[1]
    ng = 64

    tch = -(-e // (_NS * 2 * _CH)) * 2
    nch0 = max(2, 2 * round(tch * _SPLIT0 / 2))
    nch1 = tch - nch0
    nchm = max(nch0, nch1)
    cap = _NS * (nch0 + nch1) * _CH
    epad = cap - e
    src = jnp.concatenate([edge_index[0], jnp.zeros((epad,), jnp.int32)])
    dst = jnp.concatenate([edge_index[1], jnp.full((epad,), n, jnp.int32)])
    e0 = _NS * nch0 * _CH

    def _chunked(a, lo, hi, c):
        a3 = a[lo:hi].reshape(_NS, c, _CH)
        if c < nchm:
            a3 = jnp.pad(a3, ((0, 0), (0, nchm - c), (0, 0)))
        return a3

    src3 = jnp.concatenate([_chunked(src, 0, e0, nch0),
                            _chunked(src, e0, cap, nch1)], axis=0)
    dst3 = jnp.concatenate([_chunked(dst, 0, e0, nch0),
                            _chunked(dst, e0, cap, nch1)], axis=0)

    sr = -(-(n + 1) // (_NS * _CH)) * (_NS * _CH)
    z128 = jnp.zeros((_CH, d), jnp.float32)
    ones128 = jnp.ones((_CH, d), jnp.float32)

    h = _emb(x, params["emb_W"], params["emb_b"].reshape(1, -1))

    sc = _make_sc_agg(sr, nch0, nch1, d)
    deg_p = _make_sc_deg(sr, nch0, nch1, d)(dst3, z128, ones128)
    batch2d = batch.reshape(1, n)

    out = None
    for li, lp in enumerate(params["layers"]):
        bias = (lp["b_self"] + lp["b_neigh"]).reshape(1, -1)
        gam = lp["gamma"].reshape(1, -1)
        bet = lp["beta"].reshape(1, -1)
        agg_p = sc(h, src3, dst3, z128)
        if li < len(params["layers"]) - 1:
            h = _layer(h, agg_p, deg_p, lp["W_self"], lp["W_neigh"], bias, gam, bet)
        else:
            out = _final(h, agg_p, deg_p, lp["W_self"], lp["W_neigh"], bias,
                         gam, bet, batch2d, ng, params["mlp"])
    return out
